# baseline (speedup 1.0000x reference)
import functools

import jax
import jax.numpy as jnp
from jax import lax
from jax.experimental import pallas as pl
from jax.experimental.pallas import tpu as pltpu
from jax.experimental.pallas import tpu_sc as plsc

N_NODES = 10000
N_EDGES = 320000
DIM = 128

NUM_CORES = 2
NUM_SUBCORES = 16
NW = NUM_CORES * NUM_SUBCORES

CHUNK = 128
NCH = 80
IB = 8
NBLK = NCH // IB
PAD_E = NW * NCH * CHUNK
TRASH = 128
NACC = N_NODES + TRASH

INIT_PT = NACC // NUM_SUBCORES // 8 * 8
INIT_REM_OFF = INIT_PT * NUM_SUBCORES
OUT_PT = N_NODES // NUM_SUBCORES // 8 * 8
OUT_REM_OFF = OUT_PT * NUM_SUBCORES


def _sc_segment_sum(x, src_p, dst_p, zeros2d, zeros1d):
  mesh = plsc.VectorSubcoreMesh(
      core_axis_name="c", subcore_axis_name="s",
      num_cores=NUM_CORES, num_subcores=NUM_SUBCORES)

  @functools.partial(
      pl.kernel,
      out_type=[
          jax.ShapeDtypeStruct((NUM_CORES, N_NODES, DIM), jnp.float32),
          jax.ShapeDtypeStruct((NUM_CORES, NACC), jnp.float32),
      ],
      mesh=mesh,
      scratch_types=[
          pltpu.VMEM((2, IB, CHUNK), jnp.int32),
          pltpu.VMEM((2, IB, CHUNK), jnp.int32),
          pltpu.VMEM((2, CHUNK, DIM), jnp.float32),
          pltpu.VMEM((CHUNK,), jnp.float32),
          pltpu.VMEM_SHARED((NACC, DIM), jnp.float32),
          pltpu.VMEM_SHARED((NACC,), jnp.float32),
          pltpu.SemaphoreType.DMA((2,)),
          pltpu.SemaphoreType.DMA((2,)),
          pltpu.SemaphoreType.DMA((2,)),
      ],
  )
  def k(x_hbm, src_hbm, dst_hbm, z2_hbm, z1_hbm, parts_hbm, degs_hbm,
        idx_d, idx_s, rows, ones_v, acc_sh, deg_sh, gsem, ssem, isem):
    c = lax.axis_index("c")
    s = lax.axis_index("s")
    w = s * NUM_CORES + c

    for j in range(CHUNK // 16):
      ones_v[pl.ds(j * 16, 16)] = jnp.ones((16,), jnp.float32)

    pltpu.sync_copy(z2_hbm.at[pl.ds(s * INIT_PT, INIT_PT)],
                    acc_sh.at[pl.ds(s * INIT_PT, INIT_PT)])

    @pl.when(s == 0)
    def _():
      rem = NACC - INIT_REM_OFF
      pltpu.sync_copy(z2_hbm.at[pl.ds(INIT_REM_OFF, rem)],
                      acc_sh.at[pl.ds(INIT_REM_OFF, rem)])
      pltpu.sync_copy(z1_hbm, deg_sh)

    def load_block(blk, buf, sem):
      pltpu.async_copy(dst_hbm.at[w, pl.ds(blk * IB, IB)], idx_d.at[buf], sem)
      pltpu.async_copy(src_hbm.at[w, pl.ds(blk * IB, IB)], idx_s.at[buf], sem)

    def wait_block(buf, sem):
      pltpu.make_async_copy(dst_hbm.at[w, pl.ds(0, IB)], idx_d.at[buf],
                            sem).wait()
      pltpu.make_async_copy(src_hbm.at[w, pl.ds(0, IB)], idx_s.at[buf],
                            sem).wait()

    def gather(idx_buf, jl, b):
      pltpu.async_copy(x_hbm.at[idx_d.at[idx_buf, jl]], rows.at[b],
                       gsem.at[b])

    def wait_gather(b):
      pltpu.make_async_copy(x_hbm.at[pl.ds(0, CHUNK)], rows.at[b],
                            gsem.at[b]).wait()

    def scatter(idx_buf, jl, b):
      pltpu.sync_copy(rows.at[b], acc_sh.at[idx_s.at[idx_buf, jl]], add=True)
      pltpu.sync_copy(ones_v, deg_sh.at[idx_s.at[idx_buf, jl]], add=True)

    def wait_scatter(b):
      pass

    load_block(0, 0, isem.at[0])
    wait_block(0, isem.at[0])
    load_block(1, 1, isem.at[1])
    plsc.subcore_barrier()
    gather(0, 0, 0)
    gather(0, 1, 1)

    def body(k2, carry):
      for half in range(2):
        blk = 2 * k2 + half
        for jl in range(IB):
          b = jl % 2
          wait_gather(b)
          scatter(half, jl, b)

          if jl == 2:
            cond = (k2 >= 1) if half == 0 else (k2 < NBLK // 2 - 1)

            @pl.when(cond)
            def _():
              load_block(blk + 1, 1 - half, isem.at[1 - half])

          if jl < IB - 2:
            wait_scatter(b)
            gather(half, jl + 2, b)
          else:
            if jl == IB - 2:
              @pl.when(blk + 1 < NBLK)
              def _():
                wait_block(1 - half, isem.at[1 - half])

            @pl.when(blk + 1 < NBLK)
            def _():
              wait_scatter(b)
              gather(1 - half, jl + 2 - IB, b)

      return carry

    lax.fori_loop(0, NBLK // 2, body, 0)

    wait_scatter(0)
    wait_scatter(1)

    plsc.subcore_barrier()

    row0 = s * OUT_PT
    pltpu.sync_copy(acc_sh.at[pl.ds(row0, OUT_PT)],
                    parts_hbm.at[c, pl.ds(row0, OUT_PT)])

    @pl.when(s == 0)
    def _():
      rem = N_NODES - OUT_REM_OFF
      pltpu.sync_copy(acc_sh.at[pl.ds(OUT_REM_OFF, rem)],
                      parts_hbm.at[c, pl.ds(OUT_REM_OFF, rem)])
      pltpu.sync_copy(deg_sh, degs_hbm.at[c])

  return k(x, src_p, dst_p, zeros2d, zeros1d)


BLK = 2000


def _tc_combine(x, parts, degs_t, wst, bs, wnt, bn):
  def body(x_ref, p_ref, d_ref, ws_ref, bs_ref, wn_ref, bn_ref, o_ref):
    xb = x_ref[...]
    sm = jnp.dot(xb, ws_ref[...], preferred_element_type=jnp.float32)
    sm = sm + bs_ref[...]
    psum = p_ref[0] + p_ref[1]
    deg = d_ref[:, 0:1] + d_ref[:, 1:2]
    mean = psum / jnp.maximum(deg, 1.0)
    nm = jnp.dot(mean, wn_ref[...], preferred_element_type=jnp.float32)
    nm = jnp.where(deg > 0.0, nm + bn_ref[...], 0.0)
    o_ref[...] = jnp.maximum(sm + nm, 0.0)

  return pl.pallas_call(
      body,
      grid=(N_NODES // BLK,),
      in_specs=[
          pl.BlockSpec((BLK, DIM), lambda i: (i, 0)),
          pl.BlockSpec((NUM_CORES, BLK, DIM), lambda i: (0, i, 0)),
          pl.BlockSpec((BLK, NUM_CORES), lambda i: (i, 0)),
          pl.BlockSpec((DIM, DIM), lambda i: (0, 0)),
          pl.BlockSpec((1, DIM), lambda i: (0, 0)),
          pl.BlockSpec((DIM, DIM), lambda i: (0, 0)),
          pl.BlockSpec((1, DIM), lambda i: (0, 0)),
      ],
      out_specs=pl.BlockSpec((BLK, DIM), lambda i: (i, 0)),
      out_shape=jax.ShapeDtypeStruct((N_NODES, DIM), jnp.float32),
  )(x, parts, degs_t, wst, bs, wnt, bn)


def kernel(x, edge_index, W_self, b_self, W_neigh, b_neigh):
  src = edge_index[0]
  dst = edge_index[1]
  pad = PAD_E - N_EDGES
  trash = N_NODES + (jnp.arange(pad, dtype=jnp.int32) % TRASH)
  src_p = jnp.concatenate([src, trash]).reshape(NW, NCH, CHUNK)
  dst_p = jnp.concatenate(
      [dst, jnp.zeros((pad,), jnp.int32)]).reshape(NW, NCH, CHUNK)
  zeros2d = jnp.zeros((NACC, DIM), jnp.float32)
  zeros1d = jnp.zeros((NACC,), jnp.float32)
  parts, degs = _sc_segment_sum(x, src_p, dst_p, zeros2d, zeros1d)
  degs = degs[:, :N_NODES]
  return _tc_combine(x, parts, degs.T, W_self.T, b_self[None, :],
                     W_neigh.T, b_neigh[None, :])


# speedup vs baseline: 1.1311x; 1.1311x over previous
import functools

import jax
import jax.numpy as jnp
from jax import lax
from jax.experimental import pallas as pl
from jax.experimental.pallas import tpu as pltpu
from jax.experimental.pallas import tpu_sc as plsc

N_NODES = 10000
N_EDGES = 320000
DIM = 128

NUM_CORES = 2
NUM_SUBCORES = 16
NW = NUM_CORES * NUM_SUBCORES

CHUNK = 128
NCH = 80
IB = 8
NBLK = NCH // IB
PAD_E = NW * NCH * CHUNK
TRASH = 8
NACC = N_NODES + TRASH

INIT_PT = NACC // NUM_SUBCORES // 8 * 8
INIT_REM_OFF = INIT_PT * NUM_SUBCORES
OUT_PT = N_NODES // NUM_SUBCORES // 8 * 8
OUT_REM_OFF = OUT_PT * NUM_SUBCORES


def _sc_segment_sum(x, src_p, dst_p, zeros2d, zeros1d):
  mesh = plsc.VectorSubcoreMesh(
      core_axis_name="c", subcore_axis_name="s",
      num_cores=NUM_CORES, num_subcores=NUM_SUBCORES)

  @functools.partial(
      pl.kernel,
      out_type=[
          jax.ShapeDtypeStruct((NUM_CORES, N_NODES, DIM), jnp.float32),
          jax.ShapeDtypeStruct((NUM_CORES, NACC), jnp.float32),
      ],
      mesh=mesh,
      scratch_types=[
          pltpu.VMEM((2, IB, CHUNK), jnp.int32),
          pltpu.VMEM((NCH, CHUNK), jnp.int32),
          pltpu.VMEM((CHUNK, DIM), jnp.float32),
          pltpu.VMEM((CHUNK, DIM), jnp.float32),
          pltpu.VMEM((CHUNK,), jnp.float32),
          pltpu.VMEM_SHARED((NACC, DIM), jnp.float32),
          pltpu.VMEM_SHARED((NACC,), jnp.float32),
          pltpu.SemaphoreType.DMA((2,)),
          pltpu.SemaphoreType.DMA((2,)),
      ],
  )
  def k(x_hbm, src_hbm, dst_hbm, z2_hbm, z1_hbm, parts_hbm, degs_hbm,
        idx_d, idx_s, rows0, rows1, ones_v, acc_sh, deg_sh, gsem, isem):
    c = lax.axis_index("c")
    s = lax.axis_index("s")
    w = s * NUM_CORES + c
    rows = [rows0, rows1]

    for j in range(CHUNK // 16):
      ones_v[pl.ds(j * 16, 16)] = jnp.ones((16,), jnp.float32)

    pltpu.sync_copy(src_hbm.at[w], idx_s)
    pltpu.sync_copy(z2_hbm.at[pl.ds(s * INIT_PT, INIT_PT)],
                    acc_sh.at[pl.ds(s * INIT_PT, INIT_PT)])

    @pl.when(s == 0)
    def _():
      rem = NACC - INIT_REM_OFF
      pltpu.sync_copy(z2_hbm.at[pl.ds(INIT_REM_OFF, rem)],
                      acc_sh.at[pl.ds(INIT_REM_OFF, rem)])
      pltpu.sync_copy(z1_hbm, deg_sh)

    def load_block(blk, buf):
      pltpu.async_copy(dst_hbm.at[w, pl.ds(blk * IB, IB)], idx_d.at[buf],
                       isem.at[buf])

    def wait_block(buf):
      pltpu.make_async_copy(dst_hbm.at[w, pl.ds(0, IB)], idx_d.at[buf],
                            isem.at[buf]).wait()

    def gather(idx_buf, jl, b):
      pltpu.async_copy(x_hbm.at[idx_d.at[idx_buf, jl]], rows[b], gsem.at[b])

    def wait_gather(b):
      pltpu.make_async_copy(x_hbm.at[pl.ds(0, CHUNK)], rows[b],
                            gsem.at[b]).wait()

    def scatter(j, b):
      pltpu.sync_copy(rows[b], acc_sh.at[idx_s.at[j]], add=True)
      pltpu.sync_copy(ones_v, deg_sh.at[idx_s.at[j]], add=True)

    load_block(0, 0)
    wait_block(0)
    load_block(1, 1)
    plsc.subcore_barrier()
    gather(0, 0, 0)
    gather(0, 1, 1)

    def body(k2, carry):
      for half in range(2):
        blk = 2 * k2 + half
        for jl in range(IB):
          b = jl % 2
          wait_gather(b)
          scatter(blk * IB + jl, b)

          if jl == 2:
            cond = (k2 >= 1) if half == 0 else (k2 < NBLK // 2 - 1)

            @pl.when(cond)
            def _():
              load_block(blk + 1, 1 - half)

          if jl < IB - 2:
            gather(half, jl + 2, b)
          else:
            if jl == IB - 2:
              @pl.when(blk + 1 < NBLK)
              def _():
                wait_block(1 - half)

            @pl.when(blk + 1 < NBLK)
            def _():
              gather(1 - half, jl + 2 - IB, b)

      return carry

    lax.fori_loop(0, NBLK // 2, body, 0)

    plsc.subcore_barrier()

    row0 = s * OUT_PT
    pltpu.sync_copy(acc_sh.at[pl.ds(row0, OUT_PT)],
                    parts_hbm.at[c, pl.ds(row0, OUT_PT)])

    @pl.when(s == 0)
    def _():
      rem = N_NODES - OUT_REM_OFF
      pltpu.sync_copy(acc_sh.at[pl.ds(OUT_REM_OFF, rem)],
                      parts_hbm.at[c, pl.ds(OUT_REM_OFF, rem)])
      pltpu.sync_copy(deg_sh, degs_hbm.at[c])

  return k(x, src_p, dst_p, zeros2d, zeros1d)


BLK = 2000


def _tc_combine(x, parts, degs_t, wst, bs, wnt, bn):
  def body(x_ref, p_ref, d_ref, ws_ref, bs_ref, wn_ref, bn_ref, o_ref):
    xb = x_ref[...]
    sm = jnp.dot(xb, ws_ref[...], preferred_element_type=jnp.float32)
    sm = sm + bs_ref[...]
    psum = p_ref[0] + p_ref[1]
    deg = d_ref[:, 0:1] + d_ref[:, 1:2]
    mean = psum / jnp.maximum(deg, 1.0)
    nm = jnp.dot(mean, wn_ref[...], preferred_element_type=jnp.float32)
    nm = jnp.where(deg > 0.0, nm + bn_ref[...], 0.0)
    o_ref[...] = jnp.maximum(sm + nm, 0.0)

  return pl.pallas_call(
      body,
      grid=(N_NODES // BLK,),
      in_specs=[
          pl.BlockSpec((BLK, DIM), lambda i: (i, 0)),
          pl.BlockSpec((NUM_CORES, BLK, DIM), lambda i: (0, i, 0)),
          pl.BlockSpec((BLK, NUM_CORES), lambda i: (i, 0)),
          pl.BlockSpec((DIM, DIM), lambda i: (0, 0)),
          pl.BlockSpec((1, DIM), lambda i: (0, 0)),
          pl.BlockSpec((DIM, DIM), lambda i: (0, 0)),
          pl.BlockSpec((1, DIM), lambda i: (0, 0)),
      ],
      out_specs=pl.BlockSpec((BLK, DIM), lambda i: (i, 0)),
      out_shape=jax.ShapeDtypeStruct((N_NODES, DIM), jnp.float32),
  )(x, parts, degs_t, wst, bs, wnt, bn)


def kernel(x, edge_index, W_self, b_self, W_neigh, b_neigh):
  src = edge_index[0]
  dst = edge_index[1]
  pad = PAD_E - N_EDGES
  trash = N_NODES + (jnp.arange(pad, dtype=jnp.int32) % TRASH)
  src_p = (jnp.concatenate([src, trash])
           .reshape(NCH, NW, CHUNK).transpose(1, 0, 2))
  dst_p = (jnp.concatenate([dst, jnp.zeros((pad,), jnp.int32)])
           .reshape(NCH, NW, CHUNK).transpose(1, 0, 2))
  zeros2d = jnp.zeros((NACC, DIM), jnp.float32)
  zeros1d = jnp.zeros((NACC,), jnp.float32)
  parts, degs = _sc_segment_sum(x, src_p, dst_p, zeros2d, zeros1d)
  degs = degs[:, :N_NODES]
  return _tc_combine(x, parts, degs.T, W_self.T, b_self[None, :],
                     W_neigh.T, b_neigh[None, :])
